# Initial kernel scaffold; baseline (speedup 1.0000x reference)
import jax
import jax.numpy as jnp
from jax.experimental import pallas as pl


def kernel(snp, snp_ids, segment_ids, filters, W1, b1, bn1_w, bn1_b, W2, b2, bn2_w, bn2_b, Wm, bm):
    raise NotImplementedError("write your pallas kernel here")


# speedup vs baseline: 17.9007x; 17.9007x over previous
import functools

import jax
import jax.numpy as jnp
from jax import lax
from jax.experimental import pallas as pl
from jax.experimental.pallas import tpu as pltpu
from jax.experimental.pallas import tpu_sc as plsc

B = 16
N_SNPS = 100000
N_GENES = 18000
NF = 8
N_NODES = 300000

NC = 2
NS = 16
NW = NC * NS
ROWS_PER_DMA = 128
DMAS_PER_W = 75
NODES_PER_W = ROWS_PER_DMA * DMAS_PER_W
N_PAD = NW * NODES_PER_W
SENTINEL = N_GENES
G_ACC = 18048
ZROWS = G_ACC // NS
OROWS = G_ACC // NS

S_BLK = 2048


def _prep_body(snp_ref, filt_ref, out_ref):
    f = jnp.sum(filt_ref[...], axis=0) * (1.0 / NF)
    w = snp_ref[...] * f[None, :]
    eye = (lax.broadcasted_iota(jnp.int32, (B, B), 0)
           == lax.broadcasted_iota(jnp.int32, (B, B), 1)).astype(jnp.float32)
    out_ref[...] = lax.dot_general(w, eye, (((0,), (0,)), ((), ())),
                                   preferred_element_type=jnp.float32)


def _prep(snp, filters):
    grid = (pl.cdiv(N_SNPS, S_BLK),)
    return pl.pallas_call(
        _prep_body,
        grid=grid,
        in_specs=[
            pl.BlockSpec((B, S_BLK), lambda i: (0, i)),
            pl.BlockSpec((NF, S_BLK), lambda i: (0, i)),
        ],
        out_specs=pl.BlockSpec((S_BLK, B), lambda i: (i, 0)),
        out_shape=jax.ShapeDtypeStruct((N_SNPS, B), jnp.float32),
    )(snp, filters)


def _sc_body(wsnp_hbm, ids_hbm, seg_hbm, out_hbm,
             idx_v, seg_v, rows_v, zbuf_v, acc_sh, gsem):
    c = lax.axis_index("c")
    s = lax.axis_index("s")
    wid = s * NC + c

    z = jnp.zeros((16,), jnp.float32)

    def _zb(i, carry):
        zbuf_v[i, :] = z
        return carry

    lax.fori_loop(0, ZROWS, _zb, 0)
    pltpu.sync_copy(zbuf_v, acc_sh.at[pl.ds(s * ZROWS, ZROWS)])
    plsc.subcore_barrier()

    pltpu.sync_copy(ids_hbm.at[wid], idx_v)
    pltpu.sync_copy(seg_hbm.at[wid], seg_v)

    def _step(j, carry):
        pltpu.async_copy(wsnp_hbm.at[idx_v.at[j]], rows_v, gsem).wait()
        pltpu.sync_copy(rows_v, acc_sh.at[seg_v.at[j]], add=True)
        return carry

    lax.fori_loop(0, DMAS_PER_W, _step, 0)
    plsc.subcore_barrier()

    r0 = s * OROWS
    pltpu.sync_copy(acc_sh.at[pl.ds(r0, OROWS)],
                    out_hbm.at[pl.ds(c * G_ACC + r0, OROWS)])


_sc_kernel = functools.partial(
    pl.kernel,
    out_type=jax.ShapeDtypeStruct((NC * G_ACC, B), jnp.float32),
    mesh=plsc.VectorSubcoreMesh(core_axis_name="c", subcore_axis_name="s",
                                num_cores=NC, num_subcores=NS),
    scratch_types=[
        pltpu.VMEM((DMAS_PER_W, ROWS_PER_DMA), jnp.int32),
        pltpu.VMEM((DMAS_PER_W, ROWS_PER_DMA), jnp.int32),
        pltpu.VMEM((ROWS_PER_DMA, B), jnp.float32),
        pltpu.VMEM((ZROWS, B), jnp.float32),
        pltpu.VMEM_SHARED((G_ACC, B), jnp.float32),
        pltpu.SemaphoreType.DMA,
    ],
    compiler_params=pltpu.CompilerParams(use_tc_tiling_on_sc=False),
)(_sc_body)


def _mlp_body(p_ref, w1_ref, b1_ref, g1_ref, h1_ref,
              w2_ref, b2_ref, g2_ref, h2_ref, wm_ref, bm_ref, out_ref):
    psum = p_ref[0] + p_ref[1]
    inv = lax.rsqrt(jnp.float32(1.0 + 1e-5))
    h = lax.dot_general(w1_ref[...], psum, (((1,), (0,)), ((), ())),
                        preferred_element_type=jnp.float32)
    h = (h + b1_ref[...]) * inv * g1_ref[...] + h1_ref[...]
    h = jnp.maximum(h, 0.0)
    h2 = lax.dot_general(w2_ref[...], h, (((1,), (0,)), ((), ())),
                         preferred_element_type=jnp.float32)
    h2 = (h2 + b2_ref[...]) * inv * g2_ref[...] + h2_ref[...]
    feat = jnp.maximum(h2, 0.0)
    lg = lax.dot_general(wm_ref[...], feat, (((1,), (0,)), ((), ())),
                         preferred_element_type=jnp.float32)
    out_ref[...] = lg + bm_ref[...]


def _mlp(p3, W1, b1c, g1c, h1c, W2, b2c, g2c, h2c, Wm_p, bm_c):
    specs = [pl.BlockSpec((NC, N_GENES, B), lambda i: (0, 0, 0))]
    specs += [pl.BlockSpec(x.shape, lambda i, _n=len(x.shape): (0,) * _n)
              for x in (W1, b1c, g1c, h1c, W2, b2c, g2c, h2c, Wm_p, bm_c)]
    return pl.pallas_call(
        _mlp_body,
        grid=(1,),
        in_specs=specs,
        out_specs=pl.BlockSpec((1, B), lambda i: (0, 0)),
        out_shape=jax.ShapeDtypeStruct((1, B), jnp.float32),
    )(p3, W1, b1c, g1c, h1c, W2, b2c, g2c, h2c, Wm_p, bm_c)


def kernel(snp, snp_ids, segment_ids, filters, W1, b1, bn1_w, bn1_b,
           W2, b2, bn2_w, bn2_b, Wm, bm):
    wsnp = _prep(snp, filters)

    pad = N_PAD - N_NODES
    ids_p = jnp.concatenate(
        [snp_ids, jnp.zeros((pad,), jnp.int32)]).reshape(NW, DMAS_PER_W,
                                                         ROWS_PER_DMA)
    seg_p = jnp.concatenate(
        [segment_ids, jnp.full((pad,), SENTINEL, jnp.int32)]).reshape(
            NW, DMAS_PER_W, ROWS_PER_DMA)

    parts = _sc_kernel(wsnp, ids_p, seg_p)
    p3 = parts.reshape(NC, G_ACC, B)

    Wm_p = jnp.concatenate([Wm, jnp.zeros((1, 1), jnp.float32)], axis=1)
    logits = _mlp(p3, W1,
                  b1.reshape(-1, 1), bn1_w.reshape(-1, 1), bn1_b.reshape(-1, 1),
                  W2,
                  b2.reshape(-1, 1), bn2_w.reshape(-1, 1), bn2_b.reshape(-1, 1),
                  Wm_p, bm.reshape(1, 1))
    return logits.reshape(B, 1)


# speedup vs baseline: 19.3148x; 1.0790x over previous
import functools

import jax
import jax.numpy as jnp
from jax import lax
from jax.experimental import pallas as pl
from jax.experimental.pallas import tpu as pltpu
from jax.experimental.pallas import tpu_sc as plsc

B = 16
N_SNPS = 100000
N_GENES = 18000
NF = 8
N_NODES = 300000

NC = 2
NS = 16
NW = NC * NS
ROWS_PER_DMA = 128
NBUF = 4
DMAS_PER_W = 76
NODES_PER_W = ROWS_PER_DMA * DMAS_PER_W
N_PAD = NW * NODES_PER_W
N_GROUPS = DMAS_PER_W // NBUF
SENTINEL = N_GENES
G_ACC = 18048
ZROWS = G_ACC // NS
OROWS = G_ACC // NS

S_BLK = 2048


def _prep_body(snp_ref, filt_ref, out_ref):
    f = jnp.sum(filt_ref[...], axis=0) * (1.0 / NF)
    w = snp_ref[...] * f[None, :]
    eye = (lax.broadcasted_iota(jnp.int32, (B, B), 0)
           == lax.broadcasted_iota(jnp.int32, (B, B), 1)).astype(jnp.float32)
    out_ref[...] = lax.dot_general(w, eye, (((0,), (0,)), ((), ())),
                                   preferred_element_type=jnp.float32)


def _prep(snp, filters):
    grid = (pl.cdiv(N_SNPS, S_BLK),)
    return pl.pallas_call(
        _prep_body,
        grid=grid,
        in_specs=[
            pl.BlockSpec((B, S_BLK), lambda i: (0, i)),
            pl.BlockSpec((NF, S_BLK), lambda i: (0, i)),
        ],
        out_specs=pl.BlockSpec((S_BLK, B), lambda i: (i, 0)),
        out_shape=jax.ShapeDtypeStruct((N_SNPS, B), jnp.float32),
    )(snp, filters)


def _sc_body(wsnp_hbm, ids_hbm, seg_hbm, out_hbm,
             idx_v, seg_v, rows_v, zbuf_v, acc_sh,
             isem, jsem, gsems, ssems):
    c = lax.axis_index("c")
    s = lax.axis_index("s")
    wid = s * NC + c

    d_idx = pltpu.async_copy(ids_hbm.at[wid], idx_v, isem)
    d_seg = pltpu.async_copy(seg_hbm.at[wid], seg_v, jsem)

    z = jnp.zeros((16,), jnp.float32)

    def _zb(i, carry):
        zbuf_v[i, :] = z
        return carry

    lax.fori_loop(0, ZROWS, _zb, 0)
    pltpu.sync_copy(zbuf_v, acc_sh.at[pl.ds(s * ZROWS, ZROWS)])
    d_idx.wait()
    d_seg.wait()
    plsc.subcore_barrier()

    def _gather(j, b):
        return pltpu.make_async_copy(wsnp_hbm.at[idx_v.at[j]], rows_v.at[b],
                                     gsems.at[b])

    def _scatter(j, b):
        return pltpu.make_async_copy(rows_v.at[b], acc_sh.at[seg_v.at[j]],
                                     ssems.at[b])

    for b in range(NBUF):
        _gather(b, b).start()

    def _group(g, carry):
        base = g * NBUF
        for b in range(NBUF):
            _gather(base + b, b).wait()
            _scatter(base + b, b).start(add=True)
        for b in range(NBUF):
            _scatter(base + b, b).wait()

            @pl.when(g + 1 < N_GROUPS)
            def _():
                _gather(base + NBUF + b, b).start()
        return carry

    lax.fori_loop(0, N_GROUPS, _group, 0)
    plsc.subcore_barrier()

    r0 = s * OROWS
    pltpu.sync_copy(acc_sh.at[pl.ds(r0, OROWS)],
                    out_hbm.at[pl.ds(c * G_ACC + r0, OROWS)])


_sc_kernel = functools.partial(
    pl.kernel,
    out_type=jax.ShapeDtypeStruct((NC * G_ACC, B), jnp.float32),
    mesh=plsc.VectorSubcoreMesh(core_axis_name="c", subcore_axis_name="s",
                                num_cores=NC, num_subcores=NS),
    scratch_types=[
        pltpu.VMEM((DMAS_PER_W, ROWS_PER_DMA), jnp.int32),
        pltpu.VMEM((DMAS_PER_W, ROWS_PER_DMA), jnp.int32),
        pltpu.VMEM((NBUF, ROWS_PER_DMA, B), jnp.float32),
        pltpu.VMEM((ZROWS, B), jnp.float32),
        pltpu.VMEM_SHARED((G_ACC, B), jnp.float32),
        pltpu.SemaphoreType.DMA,
        pltpu.SemaphoreType.DMA,
        pltpu.SemaphoreType.DMA((NBUF,)),
        pltpu.SemaphoreType.DMA((NBUF,)),
    ],
    compiler_params=pltpu.CompilerParams(use_tc_tiling_on_sc=False),
)(_sc_body)


def _mlp_body(p_ref, w1_ref, b1_ref, g1_ref, h1_ref,
              w2_ref, b2_ref, g2_ref, h2_ref, wm_ref, bm_ref, out_ref):
    psum = p_ref[0] + p_ref[1]
    inv = lax.rsqrt(jnp.float32(1.0 + 1e-5))
    h = lax.dot_general(w1_ref[...], psum, (((1,), (0,)), ((), ())),
                        preferred_element_type=jnp.float32)
    h = (h + b1_ref[...]) * inv * g1_ref[...] + h1_ref[...]
    h = jnp.maximum(h, 0.0)
    h2 = lax.dot_general(w2_ref[...], h, (((1,), (0,)), ((), ())),
                         preferred_element_type=jnp.float32)
    h2 = (h2 + b2_ref[...]) * inv * g2_ref[...] + h2_ref[...]
    feat = jnp.maximum(h2, 0.0)
    lg = lax.dot_general(wm_ref[...], feat, (((1,), (0,)), ((), ())),
                         preferred_element_type=jnp.float32)
    out_ref[...] = lg + bm_ref[...]


def _mlp(p3, W1, b1c, g1c, h1c, W2, b2c, g2c, h2c, Wm_p, bm_c):
    specs = [pl.BlockSpec((NC, N_GENES, B), lambda i: (0, 0, 0))]
    specs += [pl.BlockSpec(x.shape, lambda i, _n=len(x.shape): (0,) * _n)
              for x in (W1, b1c, g1c, h1c, W2, b2c, g2c, h2c, Wm_p, bm_c)]
    return pl.pallas_call(
        _mlp_body,
        grid=(1,),
        in_specs=specs,
        out_specs=pl.BlockSpec((1, B), lambda i: (0, 0)),
        out_shape=jax.ShapeDtypeStruct((1, B), jnp.float32),
    )(p3, W1, b1c, g1c, h1c, W2, b2c, g2c, h2c, Wm_p, bm_c)


def kernel(snp, snp_ids, segment_ids, filters, W1, b1, bn1_w, bn1_b,
           W2, b2, bn2_w, bn2_b, Wm, bm):
    wsnp = _prep(snp, filters)

    pad = N_PAD - N_NODES
    ids_p = jnp.concatenate(
        [snp_ids, jnp.zeros((pad,), jnp.int32)]).reshape(NW, DMAS_PER_W,
                                                         ROWS_PER_DMA)
    seg_p = jnp.concatenate(
        [segment_ids, jnp.full((pad,), SENTINEL, jnp.int32)]).reshape(
            NW, DMAS_PER_W, ROWS_PER_DMA)

    parts = _sc_kernel(wsnp, ids_p, seg_p)
    p3 = parts.reshape(NC, G_ACC, B)

    Wm_p = jnp.concatenate([Wm, jnp.zeros((1, 1), jnp.float32)], axis=1)
    logits = _mlp(p3, W1,
                  b1.reshape(-1, 1), bn1_w.reshape(-1, 1), bn1_b.reshape(-1, 1),
                  W2,
                  b2.reshape(-1, 1), bn2_w.reshape(-1, 1), bn2_b.reshape(-1, 1),
                  Wm_p, bm.reshape(1, 1))
    return logits.reshape(B, 1)
